# baseline (speedup 1.0000x reference)
import jax
import jax.numpy as jnp
from jax import lax
from jax.experimental import pallas as pl
from jax.experimental.pallas import tpu as pltpu
from jax.experimental.pallas import tpu_sc as plsc

_NUM_EMBED = 1000000
_EMBED_DIM = 64
_TP_RANK = 2
_TP_SIZE = 8
_PER_PART = _NUM_EMBED // _TP_SIZE
_VS = _PER_PART * _TP_RANK
_VE = _VS + _PER_PART
_ZROW = _PER_PART
_NZROW = 64

_NC, _NS, _L = 2, 16, 16
_NW = _NC * _NS
_B, _P = 16384, 50
_N = _B * _P
_PER_W = _N // _NW
_CROWS = 512
_NCHUNK = _PER_W // _CROWS
_GD = 128
_NGD = _CROWS // _GD


def _emb_body(x_hbm, w_hbm, out_hbm, idx_v, stage, sem_g, sem_w):
  wid = lax.axis_index("s") * _NC + lax.axis_index("c")
  base = wid * _PER_W

  pltpu.sync_copy(x_hbm.at[pl.ds(base, _PER_W)], idx_v)

  def sb(t, carry):
    xv = idx_v[pl.ds(t * _L, _L)]
    m = (xv >= _VS) & (xv < _VE)
    idx_v[pl.ds(t * _L, _L)] = jnp.where(
        m, xv - _VS, _ZROW + (xv & (_NZROW - 1)))
    return carry

  lax.fori_loop(0, _PER_W // _L, sb, 0)

  def gathers(c, p):
    return [
        pltpu.make_async_copy(
            w_hbm.at[idx_v.at[pl.ds(c * _CROWS + k * _GD, _GD)]],
            stage.at[p, pl.ds(k * _GD, _GD)], sem_g.at[p])
        for k in range(_NGD)
    ]

  def write(c, p):
    return pltpu.make_async_copy(
        stage.at[p], out_hbm.at[pl.ds(base + c * _CROWS, _CROWS)], sem_w.at[p])

  for q in range(2):
    for cp in gathers(q, q):
      cp.start()

  def chunk(c, carry):
    @pl.when(c > 0)
    def _():
      write(c - 1, (c - 1) % 3).wait()

    @pl.when(c + 2 < _NCHUNK)
    def _():
      for cp in gathers(c + 2, (c + 2) % 3):
        cp.start()

    for cp in gathers(c, c % 3):
      cp.wait()
    write(c, c % 3).start()
    return carry

  lax.fori_loop(0, _NCHUNK, chunk, 0)
  write(_NCHUNK - 1, (_NCHUNK - 1) % 3).wait()


@jax.jit
def _emb(x_flat, w_ext):
  mesh = plsc.VectorSubcoreMesh(
      core_axis_name="c", subcore_axis_name="s",
      num_cores=_NC, num_subcores=_NS)
  f = pl.kernel(
      _emb_body,
      out_type=jax.ShapeDtypeStruct((_N, _EMBED_DIM), jnp.float32),
      mesh=mesh,
      scratch_types=[
          pltpu.VMEM((_PER_W,), jnp.int32),
          pltpu.VMEM((3, _CROWS, _EMBED_DIM), jnp.float32),
          pltpu.SemaphoreType.DMA((3,)),
          pltpu.SemaphoreType.DMA((3,)),
      ],
      compiler_params=pltpu.CompilerParams(
          needs_layout_passes=False, use_tc_tiling_on_sc=False),
  )
  return f(x_flat, w_ext)


def kernel(x, weight):
  w_ext = jnp.concatenate(
      [weight, jnp.zeros((_NZROW, _EMBED_DIM), jnp.float32)], axis=0)
  return _emb(x.reshape(-1).astype(jnp.int32), w_ext).reshape(
      _B, _P, _EMBED_DIM)


# speedup vs baseline: 1.7003x; 1.7003x over previous
import jax
import jax.numpy as jnp
from jax import lax
from jax.experimental import pallas as pl
from jax.experimental.pallas import tpu as pltpu
from jax.experimental.pallas import tpu_sc as plsc

_NUM_EMBED = 1000000
_EMBED_DIM = 64
_TP_RANK = 2
_TP_SIZE = 8
_PER_PART = _NUM_EMBED // _TP_SIZE
_VS = _PER_PART * _TP_RANK
_VE = _VS + _PER_PART
_ZROW = _PER_PART
_NZROW = 8192

_NC, _NS, _L = 2, 16, 16
_NW = _NC * _NS
_B, _P = 16384, 50
_N = _B * _P
_PER_W = _N // _NW
_CROWS = 512
_NCHUNK = _PER_W // _CROWS
_GD = 128
_NGD = _CROWS // _GD


def _emb_body(x_hbm, w_hbm, out_hbm, idx_v, stage, sem_g, sem_w):
  wid = lax.axis_index("s") * _NC + lax.axis_index("c")
  base = wid * _PER_W

  pltpu.sync_copy(x_hbm.at[pl.ds(base, _PER_W)], idx_v)

  def sb(t, carry):
    xv = idx_v[pl.ds(t * _L, _L)]
    m = (xv >= _VS) & (xv < _VE)
    idx_v[pl.ds(t * _L, _L)] = jnp.where(
        m, xv - _VS, _ZROW + (xv & (_NZROW - 1)))
    return carry

  lax.fori_loop(0, _PER_W // _L, sb, 0)

  def gathers(c, p):
    return [
        pltpu.make_async_copy(
            w_hbm.at[idx_v.at[pl.ds(c * _CROWS + k * _GD, _GD)]],
            stage.at[p, pl.ds(k * _GD, _GD)], sem_g.at[p])
        for k in range(_NGD)
    ]

  def write(c, p):
    return pltpu.make_async_copy(
        stage.at[p], out_hbm.at[pl.ds(base + c * _CROWS, _CROWS)], sem_w.at[p])

  for q in range(2):
    for cp in gathers(q, q):
      cp.start()

  def chunk(c, carry):
    @pl.when(c > 0)
    def _():
      write(c - 1, (c - 1) % 3).wait()

    @pl.when(c + 2 < _NCHUNK)
    def _():
      for cp in gathers(c + 2, (c + 2) % 3):
        cp.start()

    for cp in gathers(c, c % 3):
      cp.wait()
    write(c, c % 3).start()
    return carry

  lax.fori_loop(0, _NCHUNK, chunk, 0)
  write(_NCHUNK - 1, (_NCHUNK - 1) % 3).wait()


@jax.jit
def _emb(x_flat, w_ext):
  mesh = plsc.VectorSubcoreMesh(
      core_axis_name="c", subcore_axis_name="s",
      num_cores=_NC, num_subcores=_NS)
  f = pl.kernel(
      _emb_body,
      out_type=jax.ShapeDtypeStruct((_N, _EMBED_DIM), jnp.float32),
      mesh=mesh,
      scratch_types=[
          pltpu.VMEM((_PER_W,), jnp.int32),
          pltpu.VMEM((3, _CROWS, _EMBED_DIM), jnp.float32),
          pltpu.SemaphoreType.DMA((3,)),
          pltpu.SemaphoreType.DMA((3,)),
      ],
      compiler_params=pltpu.CompilerParams(
          needs_layout_passes=False, use_tc_tiling_on_sc=False),
  )
  return f(x_flat, w_ext)


def kernel(x, weight):
  w_ext = jnp.concatenate(
      [weight, jnp.zeros((_NZROW, _EMBED_DIM), jnp.float32)], axis=0)
  return _emb(x.reshape(-1).astype(jnp.int32), w_ext).reshape(
      _B, _P, _EMBED_DIM)
